# Initial kernel scaffold; baseline (speedup 1.0000x reference)
import functools
import math

import jax
import jax.numpy as jnp
from jax import lax
from jax.experimental import pallas as pl
from jax.experimental.pallas import tpu as pltpu
from jax.experimental.pallas import tpu_sc as plsc

F32 = jnp.float32

_N = 10000
_E = 320000
_D = 128

_T = 3
_CW = 16
_ROW = _D + _CW

_NC, _NS = 2, 16
_NW = _NC * _NS
_K = 128
_CH = math.ceil(_E / (_NW * _K))
_EPAD = _NW * _K * _CH
_NPAD = 10240
_RPT = _NPAD // _NS

_ROWBLK = 400


def _prologue_body(x_ref, w_ref, out_ref):
    xb = x_ref[...]
    outs = []
    lane = lax.broadcasted_iota(jnp.int32, (xb.shape[0], _CW), 1)
    for t in range(_T):
        a = lax.dot_general(xb, w_ref[t], (((1,), (0,)), ((), ())),
                            preferred_element_type=F32)
        oh = jnp.where(lane == t, 1.0, 0.0).astype(F32)
        outs.append(jnp.concatenate([a, oh], axis=1))
    out_ref[...] = jnp.stack(outs, axis=0)


def _build_table(x, w_src):
    grid = _N // _ROWBLK
    return pl.pallas_call(
        _prologue_body,
        grid=(grid,),
        in_specs=[
            pl.BlockSpec((_ROWBLK, _D), lambda i: (i, 0)),
            pl.BlockSpec((_T, _D, _D), lambda i: (0, 0, 0)),
        ],
        out_specs=pl.BlockSpec((_T, _ROWBLK, _ROW), lambda i: (0, i, 0)),
        out_shape=jax.ShapeDtypeStruct((_T, _N, _ROW), F32),
    )(x, w_src)


def _sc_body(table_hbm, gidx_hbm, dst_hbm, zeros_hbm, out_hbm,
             idxg, idxs, rows, acc, sem):
    cid = lax.axis_index("c")
    sid = lax.axis_index("s")
    wid = sid * _NC + cid

    rbase = pl.multiple_of(sid * _RPT, 8)
    pltpu.sync_copy(zeros_hbm, acc.at[pl.ds(rbase, _RPT)])
    plsc.subcore_barrier()

    ebase = wid * (_CH * _K)

    def chunk(j, carry):
        base = pl.multiple_of(ebase + j * _K, 8)
        pltpu.sync_copy(gidx_hbm.at[pl.ds(base, _K)], idxg)
        pltpu.sync_copy(dst_hbm.at[pl.ds(base, _K)], idxs)
        pltpu.async_copy(table_hbm.at[idxg], rows, sem).wait()
        pltpu.sync_copy(rows, acc.at[idxs], add=True)
        return carry

    lax.fori_loop(0, _CH, chunk, 0)
    plsc.subcore_barrier()

    pltpu.sync_copy(acc.at[pl.ds(rbase, _RPT)],
                    out_hbm.at[cid, pl.ds(rbase, _RPT)])


def _sc_scatter(table2d, gidx, dstp, zeros):
    mesh = plsc.VectorSubcoreMesh(core_axis_name="c", subcore_axis_name="s")
    kern = pl.kernel(
        _sc_body,
        out_type=jax.ShapeDtypeStruct((_NC, _NPAD, _ROW), F32),
        mesh=mesh,
        scratch_types=[
            pltpu.VMEM((_K,), jnp.int32),
            pltpu.VMEM((_K,), jnp.int32),
            pltpu.VMEM((_K, _ROW), F32),
            pltpu.VMEM_SHARED((_NPAD, _ROW), F32),
            pltpu.SemaphoreType.DMA,
        ],
    )
    return kern(table2d, gidx, dstp, zeros)


def _epilogue_body(x_ref, p_ref, wd_ref, b_ref, wu_ref, bu_ref, g_ref, be_ref,
                   out_ref):
    xb = x_ref[...]
    p = p_ref[...]
    s = p[0] + p[1]
    num = s[:, :_D]
    cnt = jnp.zeros((xb.shape[0], 1), F32)
    for t in range(_T):
        ct = s[:, _D + t:_D + t + 1]
        bt = lax.dot_general(xb, wd_ref[t], (((1,), (0,)), ((), ())),
                             preferred_element_type=F32) + b_ref[t]
        num = num + ct * bt
        cnt = cnt + ct
    agg = num / jnp.maximum(cnt, 1.0)
    h = (lax.dot_general(xb, wu_ref[:_D], (((1,), (0,)), ((), ())),
                         preferred_element_type=F32)
         + lax.dot_general(agg, wu_ref[_D:], (((1,), (0,)), ((), ())),
                           preferred_element_type=F32)
         + bu_ref[...])
    mu = jnp.mean(h, axis=1, keepdims=True)
    d = h - mu
    var = jnp.mean(d * d, axis=1, keepdims=True)
    ln = d * lax.rsqrt(var + 1e-5) * g_ref[...] + be_ref[...]
    gelu = 0.5 * ln * (1.0 + lax.erf(ln * (1.0 / math.sqrt(2.0))))
    out_ref[...] = xb + gelu


def _epilogue(x, partials, w_dst, bst, wu, bu, gamma, beta):
    grid = _N // _ROWBLK
    return pl.pallas_call(
        _epilogue_body,
        grid=(grid,),
        in_specs=[
            pl.BlockSpec((_ROWBLK, _D), lambda i: (i, 0)),
            pl.BlockSpec((_NC, _ROWBLK, _ROW), lambda i: (0, i, 0)),
            pl.BlockSpec((_T, _D, _D), lambda i: (0, 0, 0)),
            pl.BlockSpec((_T, 1, _D), lambda i: (0, 0, 0)),
            pl.BlockSpec((2 * _D, _D), lambda i: (0, 0)),
            pl.BlockSpec((1, _D), lambda i: (0, 0)),
            pl.BlockSpec((1, _D), lambda i: (0, 0)),
            pl.BlockSpec((1, _D), lambda i: (0, 0)),
        ],
        out_specs=pl.BlockSpec((_ROWBLK, _D), lambda i: (i, 0)),
        out_shape=jax.ShapeDtypeStruct((_N, _D), F32),
    )(x, partials, w_dst, bst, wu, bu, gamma, beta)


@jax.jit
def kernel(x, edge_index, edge_type, W0, b0, W1, b1, W2, b2, Wu, bu, gamma,
           beta):
    src = edge_index[0].astype(jnp.int32)
    dst = edge_index[1].astype(jnp.int32)
    et = edge_type.astype(jnp.int32)

    w_src = jnp.stack([W0[:_D], W1[:_D], W2[:_D]])
    w_dst = jnp.stack([W0[_D:], W1[_D:], W2[_D:]])
    bst = jnp.stack([b0, b1, b2])[:, None, :]

    gidx = et * _N + src
    pad = _EPAD - _E
    gidx = jnp.concatenate([gidx, jnp.zeros((pad,), jnp.int32)])
    dstp = jnp.concatenate([dst, jnp.full((pad,), _N, jnp.int32)])
    zeros = jnp.zeros((_RPT, _ROW), F32)

    table = _build_table(x, w_src).reshape(_T * _N, _ROW)
    partials = _sc_scatter(table, gidx, dstp, zeros)
    return _epilogue(x, partials, w_dst, bst, wu, bu,
                     gamma[None, :], beta[None, :])


# speedup vs baseline: 5.4090x; 5.4090x over previous
import math

import jax
import jax.numpy as jnp
from jax import lax
from jax.experimental import pallas as pl
from jax.experimental.pallas import tpu as pltpu
from jax.experimental.pallas import tpu_sc as plsc

F32 = jnp.float32

_N = 10000
_E = 320000
_D = 128

_T = 3

_NC, _NS = 2, 16
_NW = _NC * _NS
_K = 128
_CH = math.ceil(_E / (_NW * _K))
_EPAD = _NW * _K * _CH
_CHS = _EPAD // (_NS * _K)
_NPAD = 10240
_RPT = _NPAD // _NS
_HB = 768

_ROWBLK = 400


_DH = _D // _NC


def _prologue_body(x_ref, w_ref, out_ref):
    xb = x_ref[...]
    outs = []
    for c in range(_NC):
        halves = []
        for t in range(_T):
            m = lax.dot_general(xb, w_ref[t], (((1,), (0,)), ((), ())),
                                preferred_element_type=F32)
            halves.append(m[:, c * _DH:(c + 1) * _DH])
        outs.append(jnp.stack(halves, axis=0))
    out_ref[...] = jnp.stack(outs, axis=0)


def _build_table(x, w_src):
    grid = _N // _ROWBLK
    return pl.pallas_call(
        _prologue_body,
        grid=(grid,),
        in_specs=[
            pl.BlockSpec((_ROWBLK, _D), lambda i: (i, 0)),
            pl.BlockSpec((_T, _D, _D), lambda i: (0, 0, 0)),
        ],
        out_specs=pl.BlockSpec((_NC, _T, _ROWBLK, _DH), lambda i: (0, 0, i, 0)),
        out_shape=jax.ShapeDtypeStruct((_NC, _T, _N, _DH), F32),
    )(x, w_src)


def _sc_body(table_hbm, epk_hbm, zeros_hbm,
             out1_hbm, out2_hbm,
             epk_v, idxg, idxs, rows, hist, acc, sem):
    cid = lax.axis_index("c")
    sid = lax.axis_index("s")
    wid = sid * _NC + cid

    rbase = pl.multiple_of(sid * _RPT, 8)
    pltpu.sync_copy(zeros_hbm.at[pl.ds(0, _RPT)], acc.at[pl.ds(rbase, _RPT)])
    pltpu.sync_copy(zeros_hbm, hist)
    plsc.subcore_barrier()

    ebase = sid * (_CHS * _K)
    is_c0 = cid == 0

    def chunk(j, carry):
        base = pl.multiple_of(ebase + j * _K, 8)
        pltpu.sync_copy(epk_hbm.at[pl.ds(base, _K)], epk_v)
        for i in range(_K // 16):
            e = epk_v[pl.ds(i * 16, 16)]
            et = lax.bitwise_and(lax.shift_right_logical(e, 14), 3)
            gi = lax.bitwise_and(e, 0x3FFF) + (et + cid * _T) * _N
            d = lax.shift_right_logical(e, 16)
            idxg[pl.ds(i * 16, 16)] = gi
            idxs[pl.ds(i * 16, 16)] = d
            v = d * 4 + et
            cnt, last = plsc.scan_count(v)
            row = lax.shift_right_logical(v, 6)
            col = lax.bitwise_and(v, 63)
            plsc.addupdate_scatter(hist, [row, col], cnt.astype(F32),
                                   mask=jnp.logical_and(last, is_c0))
        pltpu.async_copy(table_hbm.at[idxg], rows, sem).wait()
        pltpu.sync_copy(rows, acc.at[idxs], add=True)
        return carry

    lax.fori_loop(0, _CHS, chunk, 0)
    plsc.subcore_barrier()

    pltpu.sync_copy(acc.at[pl.ds(rbase, _RPT)],
                    out1_hbm.at[cid, pl.ds(rbase, _RPT)])
    pltpu.sync_copy(hist, out2_hbm.at[wid])


def _sc_scatter(table2d, epk, zeros):
    mesh = plsc.VectorSubcoreMesh(core_axis_name="c", subcore_axis_name="s")
    kern = pl.kernel(
        _sc_body,
        out_type=(
            jax.ShapeDtypeStruct((_NC, _NPAD, _DH), F32),
            jax.ShapeDtypeStruct((_NW, _HB, _DH), F32),
        ),
        mesh=mesh,
        scratch_types=[
            pltpu.VMEM((_K,), jnp.int32),
            pltpu.VMEM((_K,), jnp.int32),
            pltpu.VMEM((_K,), jnp.int32),
            pltpu.VMEM((_K, _DH), F32),
            pltpu.VMEM((_HB, _DH), F32),
            pltpu.VMEM_SHARED((_NPAD, _DH), F32),
            pltpu.SemaphoreType.DMA,
        ],
        compiler_params=pltpu.CompilerParams(
            needs_layout_passes=False, use_tc_tiling_on_sc=False),
    )
    return kern(table2d, epk, zeros)


def _epilogue_body(x_ref, p_ref, c_ref, wd_ref, b_ref, wu_ref, bu_ref, g_ref,
                   be_ref, out_ref):
    xb = x_ref[...]
    p = p_ref[...]
    num = jnp.concatenate([p[0], p[1]], axis=1)
    cts = jnp.sum(c_ref[...], axis=0)
    cnt = jnp.zeros((xb.shape[0], 1), F32)
    for t in range(_T):
        ct = cts[:, t:t + 1]
        bt = lax.dot_general(xb, wd_ref[t], (((1,), (0,)), ((), ())),
                             preferred_element_type=F32) + b_ref[t]
        num = num + ct * bt
        cnt = cnt + ct
    agg = num / jnp.maximum(cnt, 1.0)
    h = (lax.dot_general(xb, wu_ref[:_D], (((1,), (0,)), ((), ())),
                         preferred_element_type=F32)
         + lax.dot_general(agg, wu_ref[_D:], (((1,), (0,)), ((), ())),
                           preferred_element_type=F32)
         + bu_ref[...])
    mu = jnp.mean(h, axis=1, keepdims=True)
    d = h - mu
    var = jnp.mean(d * d, axis=1, keepdims=True)
    ln = d * lax.rsqrt(var + 1e-5) * g_ref[...] + be_ref[...]
    gelu = 0.5 * ln * (1.0 + lax.erf(ln * (1.0 / math.sqrt(2.0))))
    out_ref[...] = xb + gelu


def _epilogue(x, partials, counts, w_dst, bst, wu, bu, gamma, beta):
    grid = _N // _ROWBLK
    return pl.pallas_call(
        _epilogue_body,
        grid=(grid,),
        in_specs=[
            pl.BlockSpec((_ROWBLK, _D), lambda i: (i, 0)),
            pl.BlockSpec((_NC, _ROWBLK, _DH), lambda i: (0, i, 0)),
            pl.BlockSpec((_NW, _ROWBLK, 4), lambda i: (0, i, 0)),
            pl.BlockSpec((_T, _D, _D), lambda i: (0, 0, 0)),
            pl.BlockSpec((_T, 1, _D), lambda i: (0, 0, 0)),
            pl.BlockSpec((2 * _D, _D), lambda i: (0, 0)),
            pl.BlockSpec((1, _D), lambda i: (0, 0)),
            pl.BlockSpec((1, _D), lambda i: (0, 0)),
            pl.BlockSpec((1, _D), lambda i: (0, 0)),
        ],
        out_specs=pl.BlockSpec((_ROWBLK, _D), lambda i: (i, 0)),
        out_shape=jax.ShapeDtypeStruct((_N, _D), F32),
    )(x, partials, counts, w_dst, bst, wu, bu, gamma, beta)


@jax.jit
def kernel(x, edge_index, edge_type, W0, b0, W1, b1, W2, b2, Wu, bu, gamma,
           beta):
    src = edge_index[0].astype(jnp.int32)
    dst = edge_index[1].astype(jnp.int32)
    et = edge_type.astype(jnp.int32)

    w_src = jnp.stack([W0[:_D], W1[:_D], W2[:_D]])
    w_dst = jnp.stack([W0[_D:], W1[_D:], W2[_D:]])
    bst = jnp.stack([b0, b1, b2])[:, None, :]

    pad = _EPAD - _E
    epk = src | (et << 14) | (dst << 16)
    epk = jnp.concatenate([epk, jnp.full((pad,), _N << 16, jnp.int32)])
    zeros = jnp.zeros((_HB, _DH), F32)

    table = _build_table(x, w_src).reshape(_NC * _T * _N, _DH)
    partials, hists = _sc_scatter(table, epk, zeros)
    counts = hists.reshape(_NW, _HB * _DH)[:, :4 * _N].reshape(_NW, _N, 4)
    return _epilogue(x, partials, counts, w_dst, bst, Wu, bu[None, :],
                     gamma[None, :], beta[None, :])


# speedup vs baseline: 6.8069x; 1.2584x over previous
import math

import jax
import jax.numpy as jnp
from jax import lax
from jax.experimental import pallas as pl
from jax.experimental.pallas import tpu as pltpu
from jax.experimental.pallas import tpu_sc as plsc

F32 = jnp.float32

_N = 10000
_E = 320000
_D = 128

_T = 3

_NC, _NS = 2, 16
_NW = _NC * _NS
_K = 128
_CH = math.ceil(_E / (_NW * _K))
_EPAD = _NW * _K * _CH
_CHS = _EPAD // (_NS * _K)
_NPAD = 10240
_RPT = _NPAD // _NS
_ROWBLK = 400

_DH = _D // _NC
_CW = 16
_RW = _DH + _CW


def _prologue_body(x_ref, w_ref, out_ref):
    xb = x_ref[...]
    lane = lax.broadcasted_iota(jnp.int32, (xb.shape[0], _CW), 1)
    outs = []
    for c in range(_NC):
        halves = []
        for t in range(_T):
            m = lax.dot_general(xb, w_ref[t], (((1,), (0,)), ((), ())),
                                preferred_element_type=F32)
            if c == 0:
                oh = jnp.where(lane == t, 1.0, 0.0).astype(F32)
            else:
                oh = jnp.zeros((xb.shape[0], _CW), F32)
            halves.append(jnp.concatenate([m[:, c * _DH:(c + 1) * _DH], oh],
                                          axis=1))
        outs.append(jnp.stack(halves, axis=0))
    out_ref[...] = jnp.stack(outs, axis=0)


def _build_table(x, w_src):
    grid = _N // _ROWBLK
    return pl.pallas_call(
        _prologue_body,
        grid=(grid,),
        in_specs=[
            pl.BlockSpec((_ROWBLK, _D), lambda i: (i, 0)),
            pl.BlockSpec((_T, _D, _D), lambda i: (0, 0, 0)),
        ],
        out_specs=pl.BlockSpec((_NC, _T, _ROWBLK, _RW), lambda i: (0, 0, i, 0)),
        out_shape=jax.ShapeDtypeStruct((_NC, _T, _N, _RW), F32),
    )(x, w_src)


def _sc_body(table_hbm, epk_hbm, zeros_hbm,
             out1_hbm,
             epk0, epk1, idxg0, idxs0, idxg1, idxs1, rows0, rows1,
             acc, se0, se1, sg0, sg1):
    cid = lax.axis_index("c")
    sid = lax.axis_index("s")

    rbase = pl.multiple_of(sid * _RPT, 8)
    pltpu.sync_copy(zeros_hbm, acc.at[pl.ds(rbase, _RPT)])
    plsc.subcore_barrier()

    ebase = sid * (_CHS * _K)

    def unpack(epk_v, idxg, idxs):
        for i in range(_K // 16):
            e = epk_v[pl.ds(i * 16, 16)]
            et = lax.bitwise_and(lax.shift_right_logical(e, 14), 3)
            gi = lax.bitwise_and(e, 0x3FFF) + (et + cid * _T) * _N
            d = lax.shift_right_logical(e, 16)
            idxg[pl.ds(i * 16, 16)] = gi
            idxs[pl.ds(i * 16, 16)] = d

    def ebuf(j):
        base = pl.multiple_of(ebase + j * _K, 8)
        return epk_hbm.at[pl.ds(base, _K)]

    pltpu.sync_copy(ebuf(0), epk0)
    unpack(epk0, idxg0, idxs0)
    pltpu.async_copy(table_hbm.at[idxg0], rows0, sg0)
    pltpu.async_copy(ebuf(1), epk1, se1)

    def pair(i, carry):
        a = i * 2
        pltpu.make_async_copy(ebuf(a + 1), epk1, se1).wait()
        unpack(epk1, idxg1, idxs1)
        pltpu.async_copy(ebuf(a + 2), epk0, se0)
        pltpu.make_async_copy(table_hbm.at[idxg0], rows0, sg0).wait()
        pltpu.sync_copy(rows0, acc.at[idxs0], add=True)
        pltpu.async_copy(table_hbm.at[idxg1], rows1, sg1)
        pltpu.make_async_copy(ebuf(a + 2), epk0, se0).wait()
        unpack(epk0, idxg0, idxs0)
        pltpu.async_copy(ebuf(a + 3), epk1, se1)
        pltpu.make_async_copy(table_hbm.at[idxg1], rows1, sg1).wait()
        pltpu.sync_copy(rows1, acc.at[idxs1], add=True)
        pltpu.async_copy(table_hbm.at[idxg0], rows0, sg0)
        return carry

    lax.fori_loop(0, _CHS // 2, pair, 0)

    pltpu.make_async_copy(table_hbm.at[idxg0], rows0, sg0).wait()
    pltpu.make_async_copy(ebuf(_CHS + 1), epk1, se1).wait()
    plsc.subcore_barrier()

    pltpu.sync_copy(acc.at[pl.ds(rbase, _RPT)],
                    out1_hbm.at[cid, pl.ds(rbase, _RPT)])


def _sc_scatter(table2d, epk, zeros):
    mesh = plsc.VectorSubcoreMesh(core_axis_name="c", subcore_axis_name="s")
    kern = pl.kernel(
        _sc_body,
        out_type=jax.ShapeDtypeStruct((_NC, _NPAD, _RW), F32),
        mesh=mesh,
        scratch_types=[
            pltpu.VMEM((_K,), jnp.int32),
            pltpu.VMEM((_K,), jnp.int32),
            pltpu.VMEM((_K,), jnp.int32),
            pltpu.VMEM((_K,), jnp.int32),
            pltpu.VMEM((_K,), jnp.int32),
            pltpu.VMEM((_K,), jnp.int32),
            pltpu.VMEM((_K, _RW), F32),
            pltpu.VMEM((_K, _RW), F32),
            pltpu.VMEM_SHARED((_NPAD, _RW), F32),
            pltpu.SemaphoreType.DMA,
            pltpu.SemaphoreType.DMA,
            pltpu.SemaphoreType.DMA,
            pltpu.SemaphoreType.DMA,
        ],
        compiler_params=pltpu.CompilerParams(
            needs_layout_passes=False, use_tc_tiling_on_sc=False),
    )
    return kern(table2d, epk, zeros)


def _epilogue_body(x_ref, p_ref, wd_ref, b_ref, wu_ref, bu_ref, g_ref,
                   be_ref, out_ref):
    xb = x_ref[...]
    p = p_ref[...]
    num = jnp.concatenate([p[0, :, :_DH], p[1, :, :_DH]], axis=1)
    cnt = jnp.zeros((xb.shape[0], 1), F32)
    for t in range(_T):
        ct = p[0, :, _DH + t:_DH + t + 1]
        bt = lax.dot_general(xb, wd_ref[t], (((1,), (0,)), ((), ())),
                             preferred_element_type=F32) + b_ref[t]
        num = num + ct * bt
        cnt = cnt + ct
    agg = num / jnp.maximum(cnt, 1.0)
    h = (lax.dot_general(xb, wu_ref[:_D], (((1,), (0,)), ((), ())),
                         preferred_element_type=F32)
         + lax.dot_general(agg, wu_ref[_D:], (((1,), (0,)), ((), ())),
                           preferred_element_type=F32)
         + bu_ref[...])
    mu = jnp.mean(h, axis=1, keepdims=True)
    d = h - mu
    var = jnp.mean(d * d, axis=1, keepdims=True)
    ln = d * lax.rsqrt(var + 1e-5) * g_ref[...] + be_ref[...]
    gelu = 0.5 * ln * (1.0 + lax.erf(ln * (1.0 / math.sqrt(2.0))))
    out_ref[...] = xb + gelu


def _epilogue(x, partials, w_dst, bst, wu, bu, gamma, beta):
    grid = _N // _ROWBLK
    return pl.pallas_call(
        _epilogue_body,
        grid=(grid,),
        in_specs=[
            pl.BlockSpec((_ROWBLK, _D), lambda i: (i, 0)),
            pl.BlockSpec((_NC, _ROWBLK, _RW), lambda i: (0, i, 0)),
            pl.BlockSpec((_T, _D, _D), lambda i: (0, 0, 0)),
            pl.BlockSpec((_T, 1, _D), lambda i: (0, 0, 0)),
            pl.BlockSpec((2 * _D, _D), lambda i: (0, 0)),
            pl.BlockSpec((1, _D), lambda i: (0, 0)),
            pl.BlockSpec((1, _D), lambda i: (0, 0)),
            pl.BlockSpec((1, _D), lambda i: (0, 0)),
        ],
        out_specs=pl.BlockSpec((_ROWBLK, _D), lambda i: (i, 0)),
        out_shape=jax.ShapeDtypeStruct((_N, _D), F32),
    )(x, partials, w_dst, bst, wu, bu, gamma, beta)


@jax.jit
def kernel(x, edge_index, edge_type, W0, b0, W1, b1, W2, b2, Wu, bu, gamma,
           beta):
    src = edge_index[0].astype(jnp.int32)
    dst = edge_index[1].astype(jnp.int32)
    et = edge_type.astype(jnp.int32)

    w_src = jnp.stack([W0[:_D], W1[:_D], W2[:_D]])
    w_dst = jnp.stack([W0[_D:], W1[_D:], W2[_D:]])
    bst = jnp.stack([b0, b1, b2])[:, None, :]

    pad = _EPAD + 2 * _K - _E
    epk = src | (et << 14) | (dst << 16)
    epk = jnp.concatenate([epk, jnp.full((pad,), _N << 16, jnp.int32)])
    zeros = jnp.zeros((_RPT, _RW), F32)

    table = _build_table(x, w_src).reshape(_NC * _T * _N, _RW)
    partials = _sc_scatter(table, epk, zeros)
    return _epilogue(x, partials, w_dst, bst, Wu, bu[None, :],
                     gamma[None, :], beta[None, :])
